# baseline (speedup 1.0000x reference)
import functools

import jax
import jax.numpy as jnp
from jax import lax
from jax.experimental import pallas as pl
from jax.experimental.pallas import tpu as pltpu
from jax.experimental.pallas import tpu_sc as plsc

VOCAB = 100000
EMBED_DIM = 128
BATCH = 4096
NUM_FIELDS = 26

NC = 2
NS = 16
NW = NC * NS
TOTAL = BATCH * NUM_FIELDS
CH = 128
NCH = NUM_FIELDS
NBUF = 4
NSH = 2
assert BATCH == NW * CH

_MESH = plsc.VectorSubcoreMesh(
    core_axis_name="c", subcore_axis_name="s", num_cores=NC, num_subcores=NS
)


@functools.partial(
    pl.kernel,
    out_type=jax.ShapeDtypeStruct((TOTAL, EMBED_DIM), jnp.float32),
    mesh=_MESH,
    scratch_types=[
        pltpu.VMEM((NCH, CH), jnp.int32),
        [pltpu.VMEM((CH, EMBED_DIM), jnp.float32) for _ in range(NBUF)],
        [pltpu.SemaphoreType.DMA for _ in range(NBUF)],
        [
            pltpu.VMEM_SHARED((NS * CH, EMBED_DIM), jnp.float32)
            for _ in range(NSH)
        ],
        [pltpu.SemaphoreType.DMA for _ in range(NSH)],
    ],
)
def _sc_gather(idx_hbm, table_hbm, out_hbm, idx_v, bufs, gsems, shrd, dsems):
    cid = lax.axis_index("c")
    sid = lax.axis_index("s")
    wid = cid * NS + sid
    pltpu.sync_copy(idx_hbm.at[:, pl.ds(wid * CH, CH)], idx_v)

    for b in range(NBUF):
        pltpu.async_copy(table_hbm.at[idx_v.at[b]], bufs[b], gsems[b])

    for cur in range(NCH):
        b = cur % NBUF
        p = cur % NSH
        out_slc = out_hbm.at[pl.ds(cur * BATCH + cid * (NS * CH), NS * CH)]
        if cur >= NSH:
            prev = cur - NSH
            prev_slc = out_hbm.at[
                pl.ds(prev * BATCH + cid * (NS * CH), NS * CH)
            ]
            @pl.when(sid == 0)
            def _wait_prev(p=p, prev_slc=prev_slc):
                pltpu.make_async_copy(shrd[p], prev_slc, dsems[p]).wait()

            plsc.subcore_barrier()
        pltpu.make_async_copy(
            table_hbm.at[idx_v.at[b]], bufs[b], gsems[b]
        ).wait()
        pltpu.sync_copy(bufs[b], shrd[p].at[pl.ds(sid * CH, CH)])
        if cur + NBUF < NCH:
            pltpu.async_copy(
                table_hbm.at[idx_v.at[cur + NBUF]], bufs[b], gsems[b]
            )
        plsc.subcore_barrier()

        @pl.when(sid == 0)
        def _fire_dma(p=p, out_slc=out_slc):
            pltpu.async_copy(shrd[p], out_slc, dsems[p])

    @pl.when(sid == 0)
    def _drain():
        for p in range(NSH):
            cur = NCH - NSH + p
            out_slc = out_hbm.at[
                pl.ds(cur * BATCH + cid * (NS * CH), NS * CH)
            ]
            pltpu.make_async_copy(
                shrd[cur % NSH], out_slc, dsems[cur % NSH]
            ).wait()

    plsc.subcore_barrier()


def kernel(x, W):
    idx = x.T.astype(jnp.int32)
    out = _sc_gather(idx, W)
    return out.reshape(NUM_FIELDS, BATCH, EMBED_DIM).transpose(1, 0, 2)


# speedup vs baseline: 1.0547x; 1.0547x over previous
import functools

import jax
import jax.numpy as jnp
from jax import lax
from jax.experimental import pallas as pl
from jax.experimental.pallas import tpu as pltpu
from jax.experimental.pallas import tpu_sc as plsc

VOCAB = 100000
EMBED_DIM = 128
BATCH = 4096
NUM_FIELDS = 26

NC = 2
NS = 16
NW = NC * NS
TOTAL = BATCH * NUM_FIELDS
PERW = TOTAL // NW
CH = 256
NCH = PERW // CH
NBUF = 3
assert PERW == NCH * CH

_MESH = plsc.VectorSubcoreMesh(
    core_axis_name="c", subcore_axis_name="s", num_cores=NC, num_subcores=NS
)


@functools.partial(
    pl.kernel,
    out_type=jax.ShapeDtypeStruct((TOTAL, EMBED_DIM), jnp.float32),
    mesh=_MESH,
    scratch_types=[
        pltpu.VMEM((PERW,), jnp.int32),
        [pltpu.VMEM((CH, EMBED_DIM), jnp.float32) for _ in range(NBUF)],
        [pltpu.SemaphoreType.DMA for _ in range(NBUF)],
        [pltpu.SemaphoreType.DMA for _ in range(NBUF)],
    ],
)
def _sc_gather(idx_hbm, table_hbm, out_hbm, idx_v, bufs, gsems, wsems):
    wid = lax.axis_index("c") * NS + lax.axis_index("s")
    base = wid * PERW
    pltpu.sync_copy(idx_hbm.at[pl.ds(base, PERW)], idx_v)

    def idx_slc(j):
        return idx_v.at[pl.ds(j * CH, CH)]

    def out_slc(j):
        return out_hbm.at[pl.ds(base + j * CH, CH)]

    for b in range(NBUF):
        pltpu.async_copy(table_hbm.at[idx_slc(b)], bufs[b], gsems[b])

    for cur in range(NCH):
        b = cur % NBUF
        pltpu.make_async_copy(
            table_hbm.at[idx_slc(cur)], bufs[b], gsems[b]
        ).wait()
        pltpu.async_copy(bufs[b], out_slc(cur), wsems[b])
        if cur + NBUF < NCH:
            pltpu.make_async_copy(bufs[b], out_slc(cur), wsems[b]).wait()
            pltpu.async_copy(
                table_hbm.at[idx_slc(cur + NBUF)], bufs[b], gsems[b]
            )

    for b in range(NBUF):
        pltpu.make_async_copy(
            bufs[b], out_hbm.at[pl.ds(0, CH)], wsems[b]
        ).wait()


def kernel(x, W):
    idx = x.T.astype(jnp.int32).reshape(TOTAL)
    out = _sc_gather(idx, W)
    return out.reshape(NUM_FIELDS, BATCH, EMBED_DIM).transpose(1, 0, 2)
